# Initial kernel scaffold; baseline (speedup 1.0000x reference)
import functools

import jax
import jax.numpy as jnp
from jax import lax
from jax.experimental import pallas as pl
from jax.experimental.pallas import tpu as pltpu
from jax.experimental.pallas import tpu_sc as plsc

_N = 10000
_E = 320000
_D = 128
_NPAD = 10240
_NT = 32
_EPT = _E // _NT
_K = 80

_mesh = plsc.VectorSubcoreMesh(core_axis_name="c", subcore_axis_name="s")


@functools.partial(
    pl.kernel,
    out_type=jax.ShapeDtypeStruct((2, 16, 640), jnp.float32),
    mesh=_mesh,
    scratch_types=[
        pltpu.VMEM((_EPT,), jnp.int32),
        pltpu.VMEM((_NPAD,), jnp.float32),
        pltpu.VMEM((640,), jnp.float32),
        pltpu.VMEM_SHARED((_NPAD,), jnp.float32),
    ],
)
def _hist_sc(dst_hbm, out_hbm, dst_v, hist_v, zer_v, hist_sh):
    c = lax.axis_index("c")
    s = lax.axis_index("s")
    wid = s * 2 + c
    base = wid * _EPT

    zero16 = jnp.zeros((16,), jnp.float32)

    def zloc(i, carry):
        hist_v[pl.ds(i * 16, 16)] = zero16
        return carry

    lax.fori_loop(0, _NPAD // 16, zloc, 0)

    def zz(i, carry):
        zer_v[pl.ds(i * 16, 16)] = zero16
        return carry

    lax.fori_loop(0, 640 // 16, zz, 0)
    pltpu.sync_copy(zer_v, hist_sh.at[pl.ds(s * 640, 640)])

    pltpu.sync_copy(dst_hbm.at[pl.ds(base, _EPT)], dst_v)
    ones16 = jnp.ones((16,), jnp.float32)

    def body(i, carry):
        idx = dst_v[pl.ds(i * 16, 16)]
        plsc.addupdate_scatter(hist_v, [idx], ones16)
        return carry

    lax.fori_loop(0, _EPT // 16, body, 0)

    plsc.subcore_barrier()
    pltpu.sync_copy(hist_v, hist_sh, add=True)
    plsc.subcore_barrier()
    pltpu.sync_copy(hist_sh.at[pl.ds(s * 640, 640)], out_hbm.at[c, s])


def _mm_body(x_ref, w_ref, h_ref, o_ref):
    deg = h_ref[0] + h_ref[1] + 1.0
    dinv = lax.rsqrt(deg)
    h = jnp.dot(x_ref[...], w_ref[...], preferred_element_type=jnp.float32)
    o_ref[...] = h * dinv


_BR = 1000

_mm_call = pl.pallas_call(
    _mm_body,
    grid=(_N // _BR,),
    in_specs=[
        pl.BlockSpec((_BR, _D), lambda i: (i, 0)),
        pl.BlockSpec((_D, _D), lambda i: (0, 0)),
        pl.BlockSpec((2, _BR, 1), lambda i: (0, i, 0)),
    ],
    out_specs=pl.BlockSpec((_BR, _D), lambda i: (i, 0)),
    out_shape=jax.ShapeDtypeStruct((_N, _D), jnp.float32),
)


@functools.partial(
    pl.kernel,
    out_type=jax.ShapeDtypeStruct((2, 16, 640, _D), jnp.float32),
    mesh=_mesh,
    scratch_types=[
        pltpu.VMEM((_K,), jnp.int32),
        pltpu.VMEM((_K,), jnp.int32),
        pltpu.VMEM((_K, _D), jnp.float32),
        pltpu.VMEM((128, _D), jnp.float32),
        pltpu.VMEM_SHARED((_NPAD, _D), jnp.float32),
        pltpu.SemaphoreType.DMA,
    ],
)
def _agg_sc(g_hbm, src_hbm, dst_hbm, out_hbm, si_v, di_v, rows_v, z_v, acc_sh, sem):
    c = lax.axis_index("c")
    s = lax.axis_index("s")
    wid = s * 2 + c
    base = wid * _EPT

    zero16 = jnp.zeros((16,), jnp.float32)

    def zb(i, carry):
        r = i // 8
        col = (i % 8) * 16
        z_v[r, pl.ds(col, 16)] = zero16
        return carry

    lax.fori_loop(0, 128 * (_D // 16), zb, 0)

    def zs(i, carry):
        pltpu.sync_copy(z_v, acc_sh.at[pl.ds(s * 640 + i * 128, 128)])
        return carry

    lax.fori_loop(0, 640 // 128, zs, 0)
    plsc.subcore_barrier()

    def body(i, carry):
        off = base + i * _K
        pltpu.sync_copy(src_hbm.at[pl.ds(off, _K)], si_v)
        pltpu.async_copy(g_hbm.at[si_v], rows_v, sem).wait()
        pltpu.sync_copy(dst_hbm.at[pl.ds(off, _K)], di_v)
        pltpu.sync_copy(rows_v, acc_sh.at[di_v], add=True)
        return carry

    lax.fori_loop(0, _EPT // _K, body, 0)

    plsc.subcore_barrier()

    def wb(i, carry):
        pltpu.sync_copy(
            acc_sh.at[pl.ds(s * 640 + i * 128, 128)],
            out_hbm.at[c, s, pl.ds(i * 128, 128)],
        )
        return carry

    lax.fori_loop(0, 640 // 128, wb, 0)


def _fin_body(p_ref, g_ref, h_ref, b_ref, o_ref):
    deg = h_ref[0] + h_ref[1] + 1.0
    dinv = lax.rsqrt(deg)
    tot = (p_ref[0] + p_ref[1] + g_ref[...]) * dinv + b_ref[...]
    o_ref[...] = jnp.maximum(tot, 0.0)


_fin_call = pl.pallas_call(
    _fin_body,
    grid=(_N // _BR,),
    in_specs=[
        pl.BlockSpec((2, _BR, _D), lambda i: (0, i, 0)),
        pl.BlockSpec((_BR, _D), lambda i: (i, 0)),
        pl.BlockSpec((2, _BR, 1), lambda i: (0, i, 0)),
        pl.BlockSpec((1, _D), lambda i: (0, 0)),
    ],
    out_specs=pl.BlockSpec((_BR, _D), lambda i: (i, 0)),
    out_shape=jax.ShapeDtypeStruct((_N, _D), jnp.float32),
)


def kernel(x, edge_index, W1, b1, W2, b2):
    src = edge_index[0]
    dst = edge_index[1]
    w_sum = W1 + W2
    b_sum = (b1 + b2).reshape(1, _D)

    hist = _hist_sc(dst)
    hist2 = hist.reshape(2, _NPAD)[:, :_N].reshape(2, _N, 1)
    g = _mm_call(x, w_sum, hist2)
    part = _agg_sc(g, src, dst)
    part2 = part.reshape(2, _NPAD, _D)[:, :_N]
    return _fin_call(part2, g, hist2, b_sum)


# speedup vs baseline: 19.5622x; 19.5622x over previous
import functools

import jax
import jax.numpy as jnp
from jax import lax
from jax.experimental import pallas as pl
from jax.experimental.pallas import tpu as pltpu
from jax.experimental.pallas import tpu_sc as plsc

_N = 10000
_E = 320000
_D = 128
_NPAD = 10240
_NT = 32
_EPT = _E // _NT
_K = 80

_mesh = plsc.VectorSubcoreMesh(core_axis_name="c", subcore_axis_name="s")


@functools.partial(
    pl.kernel,
    out_type=jax.ShapeDtypeStruct((2 * _NPAD, _D), jnp.float32),
    mesh=_mesh,
    scratch_types=[
        pltpu.VMEM((_K,), jnp.int32),
        pltpu.VMEM((_K, _D), jnp.float32),
        pltpu.VMEM_SHARED((_NPAD, _D), jnp.float32),
    ],
)
def _hist_sc(dst_hbm, zz_hbm, ones_hbm, out_hbm, di_v, ones_v, acc_sh):
    c = lax.axis_index("c")
    s = lax.axis_index("s")
    wid = s * 2 + c
    base = wid * _EPT

    pltpu.sync_copy(ones_hbm, ones_v)

    @pl.when(s == 0)
    def _():
        pltpu.sync_copy(zz_hbm, acc_sh)

    plsc.subcore_barrier()

    def body(i, carry):
        off = base + i * _K
        pltpu.sync_copy(dst_hbm.at[pl.ds(off, _K)], di_v)
        pltpu.sync_copy(ones_v, acc_sh.at[di_v], add=True)
        return carry

    lax.fori_loop(0, _EPT // _K, body, 0)

    plsc.subcore_barrier()

    @pl.when(s == 0)
    def _():
        pltpu.sync_copy(acc_sh, out_hbm.at[pl.ds(c * _NPAD, _NPAD)])


def _mm_body(x_ref, w_ref, h_ref, o_ref):
    deg = h_ref[0] + h_ref[1] + 1.0
    dinv = lax.rsqrt(deg)
    h = jnp.dot(x_ref[...], w_ref[...], preferred_element_type=jnp.float32)
    o_ref[...] = h * dinv


_BR = 1000

_mm_call = pl.pallas_call(
    _mm_body,
    grid=(_N // _BR,),
    in_specs=[
        pl.BlockSpec((_BR, _D), lambda i: (i, 0)),
        pl.BlockSpec((_D, _D), lambda i: (0, 0)),
        pl.BlockSpec((2, _BR, 1), lambda i: (0, i, 0)),
    ],
    out_specs=pl.BlockSpec((_BR, _D), lambda i: (i, 0)),
    out_shape=jax.ShapeDtypeStruct((_N, _D), jnp.float32),
)


@functools.partial(
    pl.kernel,
    out_type=jax.ShapeDtypeStruct((2 * _NPAD, _D), jnp.float32),
    mesh=_mesh,
    scratch_types=[
        pltpu.VMEM((_K,), jnp.int32),
        pltpu.VMEM((_K,), jnp.int32),
        pltpu.VMEM((_K, _D), jnp.float32),
        pltpu.VMEM_SHARED((_NPAD, _D), jnp.float32),
        pltpu.SemaphoreType.DMA,
    ],
)
def _agg_sc(g_hbm, src_hbm, dst_hbm, zf_hbm, out_hbm, si_v, di_v, rows_v, acc_sh, sem):
    c = lax.axis_index("c")
    s = lax.axis_index("s")
    wid = s * 2 + c
    base = wid * _EPT

    @pl.when(s == 0)
    def _():
        pltpu.sync_copy(zf_hbm, acc_sh)

    plsc.subcore_barrier()

    def body(i, carry):
        off = base + i * _K
        pltpu.sync_copy(src_hbm.at[pl.ds(off, _K)], si_v)
        pltpu.async_copy(g_hbm.at[si_v], rows_v, sem).wait()
        pltpu.sync_copy(dst_hbm.at[pl.ds(off, _K)], di_v)
        pltpu.sync_copy(rows_v, acc_sh.at[di_v], add=True)
        return carry

    lax.fori_loop(0, _EPT // _K, body, 0)

    plsc.subcore_barrier()

    @pl.when(s == 0)
    def _():
        pltpu.sync_copy(acc_sh, out_hbm.at[pl.ds(c * _NPAD, _NPAD)])


def _fin_body(p_ref, g_ref, h_ref, b_ref, o_ref):
    deg = h_ref[0] + h_ref[1] + 1.0
    dinv = lax.rsqrt(deg)
    tot = (p_ref[0] + p_ref[1] + g_ref[...]) * dinv + b_ref[...]
    o_ref[...] = jnp.maximum(tot, 0.0)


_fin_call = pl.pallas_call(
    _fin_body,
    grid=(_N // _BR,),
    in_specs=[
        pl.BlockSpec((2, _BR, _D), lambda i: (0, i, 0)),
        pl.BlockSpec((_BR, _D), lambda i: (i, 0)),
        pl.BlockSpec((2, _BR, 1), lambda i: (0, i, 0)),
        pl.BlockSpec((1, _D), lambda i: (0, 0)),
    ],
    out_specs=pl.BlockSpec((_BR, _D), lambda i: (i, 0)),
    out_shape=jax.ShapeDtypeStruct((_N, _D), jnp.float32),
)


def kernel(x, edge_index, W1, b1, W2, b2):
    src = edge_index[0]
    dst = edge_index[1]
    w_sum = W1 + W2
    b_sum = (b1 + b2).reshape(1, _D)

    zf = jnp.zeros((_NPAD, _D), jnp.float32)
    ones_blk = jnp.ones((_K, _D), jnp.float32)

    hist = _hist_sc(dst, zf, ones_blk)
    hist2 = hist.reshape(2, _NPAD, _D)[:, :_N, :1]
    g = _mm_call(x, w_sum, hist2)
    part = _agg_sc(g, src, dst, zf)
    part2 = part.reshape(2, _NPAD, _D)[:, :_N]
    return _fin_call(part2, g, hist2, b_sum)
